# baseline (speedup 1.0000x reference)
import functools
import math

import jax
import jax.numpy as jnp
from jax import lax
from jax.experimental import pallas as pl
from jax.experimental.pallas import tpu as pltpu
from jax.experimental.pallas import tpu_sc as plsc

D_MODEL = 64
SCALE = math.sqrt(D_MODEL)


def _tc_pack(tableT):
    Dm, V = tableT.shape
    grid = (V + 511) // 512

    def body(x1_ref, x2_ref, o_ref):
        eye = jnp.eye(Dm, dtype=jnp.float32) * SCALE
        t1 = jax.lax.dot_general(x1_ref[...], eye, (((0,), (0,)), ((), ())))
        t2 = jax.lax.dot_general(x2_ref[...], eye, (((0,), (0,)), ((), ())))
        o_ref[...] = jnp.concatenate([t1, t2], axis=1)

    return pl.pallas_call(
        body,
        grid=(grid,),
        in_specs=[
            pl.BlockSpec((Dm, 256),
                         lambda j: (0, jnp.minimum(2 * j, (V - 1) // 256))),
            pl.BlockSpec((Dm, 256),
                         lambda j: (0, jnp.minimum(2 * j + 1, (V - 1) // 256))),
        ],
        out_specs=pl.BlockSpec((256, 128), lambda j: (j, 0)),
        out_shape=jax.ShapeDtypeStruct((grid * 256, 128), jnp.float32),
    )(tableT, tableT)


def _build(B, H):
    NW = 32
    assert B == 128 * NW and H % 2 == 0
    mesh = plsc.VectorSubcoreMesh(core_axis_name="c", subcore_axis_name="s")

    @functools.partial(
        pl.kernel,
        mesh=mesh,
        out_type=jax.ShapeDtypeStruct((H * 8, NW, 8 * 128), jnp.float32),
        compiler_params=pltpu.CompilerParams(
            use_tc_tiling_on_sc=False, needs_layout_passes=False,
            disable_bounds_checks=True),
        scratch_types=[
            pltpu.VMEM((H, 128), jnp.int32),
            pltpu.VMEM((2, 128, D_MODEL), jnp.float32),
            pltpu.VMEM((2, 8, 1, 8 * 128), jnp.float32),
            pltpu.SemaphoreType.DMA,
            pltpu.SemaphoreType.DMA,
            pltpu.SemaphoreType.DMA,
            pltpu.SemaphoreType.DMA,
        ],
    )
    def emb(x4_hbm, tab_hbm, out_hbm, idx_v, rows_v, outb_v,
            sem0, sem1, osem0, osem1):
        cid = lax.axis_index("c")
        sid = lax.axis_index("s")
        wid = sid * 2 + cid
        iota = lax.iota(jnp.int32, 16)

        pltpu.sync_copy(x4_hbm.at[pl.ds(pl.multiple_of(wid * H, 8), H)],
                        idx_v)

        def prep_r(r, c):
            def prep_q(q, c2):
                v = idx_v[r, pl.ds(16 * q, 16)]
                idx_v[r, pl.ds(16 * q, 16)] = (
                    (v & -512) + ((v & 255) << 1) + ((v >> 8) & 1))
                return c2
            return lax.fori_loop(0, 8, prep_q, c)

        lax.fori_loop(0, H, prep_r, 0)

        def out_dst(h):
            return out_hbm.at[pl.ds(pl.multiple_of(8 * h, 8), 8),
                              pl.ds(wid, 1), pl.ds(0, 8 * 128)]

        def gather(h, buf, sem):
            return pltpu.async_copy(
                tab_hbm.at[idx_v.at[h]], rows_v.at[buf], sem)

        def gather_wait(buf, sem):
            pltpu.make_async_copy(
                tab_hbm.at[idx_v.at[0]], rows_v.at[buf], sem).wait()

        def compute(h, buf, osem, first):
            @pl.when(jnp.logical_not(first))
            def _():
                pltpu.make_async_copy(
                    out_dst(0), outb_v.at[buf], osem).wait()
            rows = rows_v.at[buf]
            outb = outb_v.at[buf]
            brows = [16 * q + iota for q in range(8)]

            @plsc.parallel_loop(0, 8, unroll=2)
            def i_body(i):
                for s in range(8):
                    d = jnp.broadcast_to(8 * i + s, (16,))
                    for q in range(8):
                        outb[i, 0, pl.ds(128 * s + 16 * q, 16)] = (
                            plsc.load_gather(rows, [brows[q], d]))

            pltpu.async_copy(outb, out_dst(h), osem)

        gather(0, 0, sem0)

        def pair(t, c):
            h0 = 2 * t
            gather(h0 + 1, 1, sem1)
            gather_wait(0, sem0)
            compute(h0, 0, osem0, t == 0)

            @pl.when(t < H // 2 - 1)
            def _():
                gather(h0 + 2, 0, sem0)
            gather_wait(1, sem1)
            compute(h0 + 1, 1, osem1, t == 0)
            return c

        lax.fori_loop(0, H // 2, pair, 0)
        for buf, osem in ((0, osem0), (1, osem1)):
            pltpu.make_async_copy(out_dst(0), outb_v.at[buf], osem).wait()

    return emb


def kernel(x, table):
    B, H = x.shape
    NW = B // 128
    tabP = _tc_pack(table.T)
    tabL = tabP.reshape(2 * tabP.shape[0], D_MODEL)
    x4 = x.T.reshape(H, NW, 128).transpose(1, 0, 2).reshape(NW * H, 128)
    out3 = _build(B, H)(x4, tabL)
    out5 = out3.reshape(H, 8, NW, 8, 128)
    return out5.transpose(2, 4, 0, 1, 3).reshape(B, H, D_MODEL)


# speedup vs baseline: 2.1816x; 2.1816x over previous
import functools
import math

import jax
import jax.numpy as jnp
from jax import lax
from jax.experimental import pallas as pl
from jax.experimental.pallas import tpu as pltpu
from jax.experimental.pallas import tpu_sc as plsc

D_MODEL = 64
SCALE = math.sqrt(D_MODEL)


def _tc_pack(tableT):
    Dm, V = tableT.shape
    W = 8192
    grid = (V + W - 1) // W

    def body(x_ref, o_ref):
        for k in range(W // 512):
            t = x_ref[:, 512 * k:512 * (k + 1)].T * SCALE
            o_ref[256 * k:256 * (k + 1), :] = jnp.concatenate(
                [t[0:256], t[256:512]], axis=1)

    return pl.pallas_call(
        body,
        grid=(grid,),
        in_specs=[pl.BlockSpec((Dm, W), lambda j: (0, j))],
        out_specs=pl.BlockSpec((W // 2, 128), lambda j: (j, 0)),
        out_shape=jax.ShapeDtypeStruct((grid * W // 2, 128), jnp.float32),
    )(tableT)


def _build(B, H):
    NW = 32
    assert B == 128 * NW and H % 2 == 0
    mesh = plsc.VectorSubcoreMesh(core_axis_name="c", subcore_axis_name="s")

    @functools.partial(
        pl.kernel,
        mesh=mesh,
        out_type=jax.ShapeDtypeStruct((H * 8, NW, 8 * 128), jnp.float32),
        compiler_params=pltpu.CompilerParams(
            use_tc_tiling_on_sc=False, needs_layout_passes=False,
            disable_bounds_checks=True),
        scratch_types=[
            pltpu.VMEM((H, 128), jnp.int32),
            pltpu.VMEM((2, 128, D_MODEL), jnp.float32),
            pltpu.VMEM((2, 8, 1, 8 * 128), jnp.float32),
            pltpu.SemaphoreType.DMA,
            pltpu.SemaphoreType.DMA,
            pltpu.SemaphoreType.DMA,
            pltpu.SemaphoreType.DMA,
        ],
    )
    def emb(x4_hbm, tab_hbm, out_hbm, idx_v, rows_v, outb_v,
            sem0, sem1, osem0, osem1):
        cid = lax.axis_index("c")
        sid = lax.axis_index("s")
        wid = sid * 2 + cid
        iota = lax.iota(jnp.int32, 16)

        pltpu.sync_copy(x4_hbm.at[pl.ds(pl.multiple_of(wid * H, 8), H)],
                        idx_v)

        def prep_r(r, c):
            def prep_q(q, c2):
                v = idx_v[r, pl.ds(16 * q, 16)]
                idx_v[r, pl.ds(16 * q, 16)] = (
                    (v & -512) + ((v & 255) << 1) + ((v >> 8) & 1))
                return c2
            return lax.fori_loop(0, 8, prep_q, c)

        lax.fori_loop(0, H, prep_r, 0)

        def out_dst(h):
            return out_hbm.at[pl.ds(pl.multiple_of(8 * h, 8), 8),
                              pl.ds(wid, 1), pl.ds(0, 8 * 128)]

        def gather(h, buf, sem):
            return pltpu.async_copy(
                tab_hbm.at[idx_v.at[h]], rows_v.at[buf], sem)

        def gather_wait(buf, sem):
            pltpu.make_async_copy(
                tab_hbm.at[idx_v.at[0]], rows_v.at[buf], sem).wait()

        def compute(h, buf, osem, first):
            @pl.when(jnp.logical_not(first))
            def _():
                pltpu.make_async_copy(
                    out_dst(0), outb_v.at[buf], osem).wait()
            rows = rows_v.at[buf]
            outb = outb_v.at[buf]
            brows = [16 * q + iota for q in range(8)]

            @plsc.parallel_loop(0, D_MODEL, unroll=8)
            def d_body(d):
                i2 = d >> 3
                base = (d & 7) * 128
                dv = jnp.broadcast_to(d, (16,))
                for q in range(8):
                    outb[i2, 0, pl.ds(base + 16 * q, 16)] = (
                        plsc.load_gather(rows, [brows[q], dv]))

            pltpu.async_copy(outb, out_dst(h), osem)

        gather(0, 0, sem0)

        def pair(t, c):
            h0 = 2 * t
            gather(h0 + 1, 1, sem1)
            gather_wait(0, sem0)
            compute(h0, 0, osem0, t == 0)

            @pl.when(t < H // 2 - 1)
            def _():
                gather(h0 + 2, 0, sem0)
            gather_wait(1, sem1)
            compute(h0 + 1, 1, osem1, t == 0)
            return c

        lax.fori_loop(0, H // 2, pair, 0)
        for buf, osem in ((0, osem0), (1, osem1)):
            pltpu.make_async_copy(out_dst(0), outb_v.at[buf], osem).wait()

    return emb


def kernel(x, table):
    B, H = x.shape
    NW = B // 128
    tabP = _tc_pack(table.T)
    tabL = tabP.reshape(2 * tabP.shape[0], D_MODEL)
    x4 = x.T.reshape(H, NW, 128).transpose(1, 0, 2).reshape(NW * H, 128)
    out3 = _build(B, H)(x4, tabL)
    out5 = out3.reshape(H, 8, NW, 8, 128)
    return out5.transpose(2, 4, 0, 1, 3).reshape(B, H, D_MODEL)
